# baseline (speedup 1.0000x reference)
import jax
import jax.numpy as jnp
from jax import lax
from jax.experimental import pallas as pl
from jax.experimental.pallas import tpu as pltpu
from jax.experimental.pallas import tpu_sc as plsc

_NC, _NS, _L = 2, 16, 16
_NW = _NC * _NS

_N = 32768
_D = 1024
_RW = _N // _NW
_G = 32
_T = 32
_BB = _RW // _G
_TB = _RW // _T


def _sc_body(f_hbm, t_hbm, midx_hbm, uidx_hbm, cnt_hbm, o_hbm,
             idxm_v, idxu_v, cnt_v, tok, ubuf,
             sem_g0, sem_g1, sem_s0, sem_s1, sem_t):
    wid = lax.axis_index("c") * _NS + lax.axis_index("s")
    wbase = wid * _RW

    pltpu.sync_copy(cnt_hbm.at[pl.ds(wid * _L, _L)], cnt_v)
    mc = cnt_v[pl.ds(0, _L)][0]
    uc = _RW - mc
    pltpu.sync_copy(midx_hbm.at[pl.ds(wbase, _RW)], idxm_v)
    pltpu.sync_copy(uidx_hbm.at[pl.ds(wbase, _RW)], idxu_v)

    tok_fill = pltpu.make_async_copy(t_hbm, tok, sem_t)
    tok_fill.start()
    tok_fill.wait()

    for tb in range(_TB):
        @pl.when(tb * _T < mc)
        def _(tb=tb):
            pltpu.make_async_copy(
                tok, o_hbm.at[idxm_v.at[pl.ds(tb * _T, _T)]], sem_t
            ).start()

    sem_g = (sem_g0, sem_g1)
    sem_s = (sem_s0, sem_s1)

    def gdesc(b):
        s = b % 2
        return pltpu.make_async_copy(
            f_hbm.at[idxu_v.at[pl.ds(b * _G, _G)]], ubuf.at[s], sem_g[s]
        )

    def sdesc(b):
        s = b % 2
        return pltpu.make_async_copy(
            ubuf.at[s], o_hbm.at[idxu_v.at[pl.ds(b * _G, _G)]], sem_s[s]
        )

    for b in range(_BB + 1):
        if b >= 2 and b - 2 < _BB:
            @pl.when((b - 2) * _G < uc)
            def _(b=b):
                sdesc(b - 2).wait()
        if b < _BB:
            @pl.when(b * _G < uc)
            def _(b=b):
                gdesc(b).start()
        if b >= 1:
            bp = b - 1

            @pl.when(bp * _G < uc)
            def _(bp=bp):
                gdesc(bp).wait()
                sdesc(bp).start()

    @pl.when((_BB - 1) * _G < uc)
    def _():
        sdesc(_BB - 1).wait()

    for tb in range(_TB):
        @pl.when(tb * _T < mc)
        def _(tb=tb):
            pltpu.make_async_copy(
                tok, o_hbm.at[idxm_v.at[pl.ds(tb * _T, _T)]], sem_t
            ).wait()


def _index_prep(mask):
    blk = mask.reshape(_NW, _RW)
    key = jnp.where(blk, jnp.int32(0), jnp.int32(1))
    offs = (jnp.arange(_NW, dtype=jnp.int32) * _RW)[:, None]
    r = jnp.arange(_RW, dtype=jnp.int32)[None, :]
    perm_m = jnp.argsort(key, axis=1, stable=True).astype(jnp.int32) + offs
    perm_u = jnp.argsort(1 - key, axis=1, stable=True).astype(jnp.int32) + offs
    mc = jnp.sum(key == 0, axis=1, dtype=jnp.int32)
    midx = jnp.where(r < mc[:, None], perm_m, perm_m[:, :1]).reshape(-1)
    uidx = jnp.where(r < (_RW - mc)[:, None], perm_u, perm_u[:, :1]).reshape(-1)
    cnts = jnp.zeros((_NW, _L), jnp.int32).at[:, 0].set(mc).reshape(-1)
    return midx, uidx, cnts


def kernel(features, mask, mask_token):
    B, S, D = features.shape
    N = B * S
    f2 = features.reshape(N, D)
    t2 = jnp.broadcast_to(mask_token.reshape(1, D), (_T, D))
    midx, uidx, cnts = _index_prep(mask)

    mesh = plsc.VectorSubcoreMesh(core_axis_name="c", subcore_axis_name="s")
    run = pl.kernel(
        _sc_body,
        out_type=jax.ShapeDtypeStruct((N, D), features.dtype),
        mesh=mesh,
        scratch_types=[
            pltpu.VMEM((_RW,), jnp.int32),
            pltpu.VMEM((_RW,), jnp.int32),
            pltpu.VMEM((_L,), jnp.int32),
            pltpu.VMEM((_T, _D), jnp.float32),
            pltpu.VMEM((2, _G, _D), jnp.float32),
            pltpu.SemaphoreType.DMA,
            pltpu.SemaphoreType.DMA,
            pltpu.SemaphoreType.DMA,
            pltpu.SemaphoreType.DMA,
            pltpu.SemaphoreType.DMA,
        ],
    )
    out = run(f2, t2, midx, uidx, cnts)
    return out.reshape(B, S, D)


# speedup vs baseline: 1.0300x; 1.0300x over previous
import jax
import jax.numpy as jnp
from jax import lax
from jax.experimental import pallas as pl
from jax.experimental.pallas import tpu as pltpu
from jax.experimental.pallas import tpu_sc as plsc

_NC, _NS, _L = 2, 16, 16
_NW = _NC * _NS

_N = 32768
_D = 1024
_RW = _N // _NW
_G = 32
_T = 32
_BB = _RW // _G
_TB = _RW // _T


def _sc_body(f_hbm, t_hbm, midx_hbm, uidx_hbm, cnt_hbm, o_hbm,
             idxm_v, idxu_v, cnt_v, tok, ubuf,
             sem_g0, sem_g1, sem_s0, sem_s1, sem_t):
    wid = lax.axis_index("c") * _NS + lax.axis_index("s")
    wbase = wid * _RW

    pltpu.sync_copy(cnt_hbm.at[pl.ds(wid * _L, _L)], cnt_v)
    mc = cnt_v[pl.ds(0, _L)][0]
    uc = _RW - mc
    pltpu.sync_copy(midx_hbm.at[pl.ds(wbase, _RW)], idxm_v)
    pltpu.sync_copy(uidx_hbm.at[pl.ds(wbase, _RW)], idxu_v)

    tok_fill = pltpu.make_async_copy(t_hbm, tok, sem_t)
    tok_fill.start()
    tok_fill.wait()

    for tb in range(_TB):
        @pl.when(tb * _T < mc)
        def _(tb=tb):
            pltpu.make_async_copy(
                tok, o_hbm.at[idxm_v.at[pl.ds(tb * _T, _T)]], sem_t
            ).start()

    sem_g = (sem_g0, sem_g1)
    sem_s = (sem_s0, sem_s1)

    def gdesc(b):
        s = b % 2
        return pltpu.make_async_copy(
            f_hbm.at[idxu_v.at[pl.ds(b * _G, _G)]], ubuf.at[s], sem_g[s]
        )

    def sdesc(b):
        s = b % 2
        return pltpu.make_async_copy(
            ubuf.at[s], o_hbm.at[idxu_v.at[pl.ds(b * _G, _G)]], sem_s[s]
        )

    for b in range(_BB + 1):
        if b >= 2 and b - 2 < _BB:
            @pl.when((b - 2) * _G < uc)
            def _(b=b):
                sdesc(b - 2).wait()
        if b < _BB:
            @pl.when(b * _G < uc)
            def _(b=b):
                gdesc(b).start()
        if b >= 1:
            bp = b - 1

            @pl.when(bp * _G < uc)
            def _(bp=bp):
                gdesc(bp).wait()
                sdesc(bp).start()

    @pl.when((_BB - 1) * _G < uc)
    def _():
        sdesc(_BB - 1).wait()

    for tb in range(_TB):
        @pl.when(tb * _T < mc)
        def _(tb=tb):
            pltpu.make_async_copy(
                tok, o_hbm.at[idxm_v.at[pl.ds(tb * _T, _T)]], sem_t
            ).wait()


def _index_prep(mask):
    blk = mask.reshape(_NW, _RW)
    key = jnp.where(blk, jnp.int32(0), jnp.int32(1))
    offs = (jnp.arange(_NW, dtype=jnp.int32) * _RW)[:, None]
    r = jnp.arange(_RW, dtype=jnp.int32)[None, :]
    perm_m = jnp.argsort(key, axis=1, stable=True).astype(jnp.int32) + offs
    perm_u = jnp.flip(perm_m, axis=1)
    mc = jnp.sum(key == 0, axis=1, dtype=jnp.int32)
    midx = jnp.where(r < mc[:, None], perm_m, perm_m[:, :1]).reshape(-1)
    uidx = jnp.where(r < (_RW - mc)[:, None], perm_u, perm_u[:, :1]).reshape(-1)
    cnts = jnp.zeros((_NW, _L), jnp.int32).at[:, 0].set(mc).reshape(-1)
    return midx, uidx, cnts


def kernel(features, mask, mask_token):
    B, S, D = features.shape
    N = B * S
    f2 = features.reshape(N, D)
    t2 = jnp.broadcast_to(mask_token.reshape(1, D), (_T, D))
    midx, uidx, cnts = _index_prep(mask)

    mesh = plsc.VectorSubcoreMesh(core_axis_name="c", subcore_axis_name="s")
    run = pl.kernel(
        _sc_body,
        out_type=jax.ShapeDtypeStruct((N, D), features.dtype),
        mesh=mesh,
        scratch_types=[
            pltpu.VMEM((_RW,), jnp.int32),
            pltpu.VMEM((_RW,), jnp.int32),
            pltpu.VMEM((_L,), jnp.int32),
            pltpu.VMEM((_T, _D), jnp.float32),
            pltpu.VMEM((2, _G, _D), jnp.float32),
            pltpu.SemaphoreType.DMA,
            pltpu.SemaphoreType.DMA,
            pltpu.SemaphoreType.DMA,
            pltpu.SemaphoreType.DMA,
            pltpu.SemaphoreType.DMA,
        ],
    )
    out = run(f2, t2, midx, uidx, cnts)
    return out.reshape(B, S, D)


# speedup vs baseline: 1.5663x; 1.5206x over previous
import functools

import jax
import jax.numpy as jnp
from jax.experimental import pallas as pl
from jax.experimental.pallas import tpu as pltpu


def _body(N, D, R, K, f_ref, m_ref, t_ref, o_ref, in_buf, out_buf, in_sem, out_sem):
    steps = N // R

    def in_dma(chunk, slot):
        return pltpu.make_async_copy(
            f_ref.at[pl.ds(chunk * R, R), :], in_buf.at[slot], in_sem.at[slot]
        )

    def out_dma(chunk, slot):
        return pltpu.make_async_copy(
            out_buf.at[slot], o_ref.at[pl.ds(chunk * R, R), :], out_sem.at[slot]
        )

    for j in range(K):
        in_dma(j, j).start()

    for i in range(steps):
        slot = i % K
        in_dma(i, slot).wait()
        if i >= K:
            out_dma(i - K, slot).wait()
        m = m_ref[:, pl.ds(i * R, R)].astype(jnp.int32).reshape(R, 1) != 0
        out_buf[slot] = jnp.where(m, t_ref[...], in_buf[slot])
        out_dma(i, slot).start()
        if i + K < steps:
            in_dma(i + K, slot).start()

    for j in range(steps - K, steps):
        out_dma(j, j % K).wait()


def kernel(features, mask, mask_token):
    B, S, D = features.shape
    N = B * S
    R = 512
    K = 6
    f2 = features.reshape(N, D)
    m2 = mask.reshape(1, N)
    t2 = mask_token.reshape(1, D)
    out = pl.pallas_call(
        functools.partial(_body, N, D, R, K),
        in_specs=[
            pl.BlockSpec(memory_space=pl.ANY),
            pl.BlockSpec(memory_space=pltpu.VMEM),
            pl.BlockSpec(memory_space=pltpu.VMEM),
        ],
        out_specs=pl.BlockSpec(memory_space=pl.ANY),
        out_shape=jax.ShapeDtypeStruct((N, D), features.dtype),
        scratch_shapes=[
            pltpu.VMEM((K, R, D), features.dtype),
            pltpu.VMEM((K, R, D), features.dtype),
            pltpu.SemaphoreType.DMA((K,)),
            pltpu.SemaphoreType.DMA((K,)),
        ],
    )(f2, m2, t2)
    return out.reshape(B, S, D)


# speedup vs baseline: 1.5822x; 1.0101x over previous
import jax
import jax.numpy as jnp
from jax.experimental import pallas as pl


def _body(f_ref, m_ref, t_ref, o_ref):
    R = f_ref.shape[0]
    m = m_ref[0].astype(jnp.int32).reshape(R, 1) != 0
    o_ref[...] = jnp.where(m, t_ref[...], f_ref[...])


def kernel(features, mask, mask_token):
    B, S, D = features.shape
    N = B * S
    R = 2048
    f2 = features.reshape(N, D)
    m3 = mask.reshape(N // R, 1, R)
    t2 = mask_token.reshape(1, D)
    grid = (N // R,)
    out = pl.pallas_call(
        _body,
        grid=grid,
        in_specs=[
            pl.BlockSpec((R, D), lambda i: (i, 0)),
            pl.BlockSpec((1, 1, R), lambda i: (i, 0, 0)),
            pl.BlockSpec((1, D), lambda i: (0, 0)),
        ],
        out_specs=pl.BlockSpec((R, D), lambda i: (i, 0)),
        out_shape=jax.ShapeDtypeStruct((N, D), features.dtype),
    )(f2, m3, t2)
    return out.reshape(B, S, D)
